# baseline (speedup 1.0000x reference)
import functools

import jax
import jax.numpy as jnp
import numpy as np
from jax import lax
from jax.experimental import pallas as pl
from jax.experimental.pallas import tpu as pltpu
from jax.experimental.pallas import tpu_sc as plsc

NUM_TIMESTEPS = 1000


def _make_tables() -> tuple[np.ndarray, np.ndarray]:
    s = np.float32(0.0001)
    x = np.linspace(0.0, float(NUM_TIMESTEPS), NUM_TIMESTEPS + 1, dtype=np.float32)
    acp = np.cos((x / NUM_TIMESTEPS + s) / (1 + s) * np.float32(np.pi) * 0.5,
                 dtype=np.float32) ** 2
    acp = acp / acp[0]
    betas = (1.0 - acp[1:] / acp[:-1]).astype(np.float32)
    betas = np.clip(betas, np.float32(0.02), np.float32(0.02))
    alphas = (1.0 - betas).astype(np.float32)
    acp2 = np.cumprod(alphas, dtype=np.float32)
    return np.sqrt(acp2), np.sqrt(np.float32(1.0) - acp2)


_TABLE_A, _TABLE_B = _make_tables()


@functools.cache
def _make_sc_gather(batch: int):
    info = plsc.get_sparse_core_info()
    num_cores = info.num_cores
    num_workers = num_cores * info.num_subcores
    b_per_w = batch // num_workers
    mesh = plsc.VectorSubcoreMesh(core_axis_name="c", subcore_axis_name="s")

    @functools.partial(
        pl.kernel,
        mesh=mesh,
        out_type=jax.ShapeDtypeStruct((2 * batch,), jnp.float32),
        scratch_types=[
            pltpu.VMEM((b_per_w,), jnp.int32),
            pltpu.VMEM((b_per_w,), jnp.float32),
            pltpu.VMEM((b_per_w,), jnp.float32),
            pltpu.SemaphoreType.DMA,
            pltpu.SemaphoreType.DMA,
        ],
    )
    def gather(ta_hbm, tb_hbm, ts_hbm, out_hbm, idx_v, a_v, b_v, sem_a, sem_b):
        wid = lax.axis_index("s") * num_cores + lax.axis_index("c")
        base = wid * b_per_w
        pltpu.sync_copy(ts_hbm.at[pl.ds(base, b_per_w)], idx_v)
        ca = pltpu.async_copy(ta_hbm.at[idx_v], a_v, sem_a)
        cb = pltpu.async_copy(tb_hbm.at[idx_v], b_v, sem_b)
        ca.wait()
        cb.wait()
        pltpu.sync_copy(a_v, out_hbm.at[pl.ds(base, b_per_w)])
        pltpu.sync_copy(b_v, out_hbm.at[pl.ds(batch + base, b_per_w)])

    return gather


def _combine_body(coef_ref, x_ref, n_ref, o_ref):
    batch = x_ref.shape[1]
    c = coef_ref[...]
    a = c[0:batch].reshape(1, batch)
    b = c[batch:2 * batch].reshape(1, batch)
    o_ref[...] = a * x_ref[...] + b * n_ref[...]


def _combine(coefs, xt, nt, block_r: int, interpret: bool = False):
    rows, batch = xt.shape
    return pl.pallas_call(
        _combine_body,
        grid=(rows // block_r,),
        in_specs=[
            pl.BlockSpec((2 * batch,), lambda i: (0,)),
            pl.BlockSpec((block_r, batch), lambda i: (i, 0)),
            pl.BlockSpec((block_r, batch), lambda i: (i, 0)),
        ],
        out_specs=pl.BlockSpec((block_r, batch), lambda i: (i, 0)),
        out_shape=jax.ShapeDtypeStruct((rows, batch), jnp.float32),
        interpret=interpret,
    )(coefs, xt, nt)


def kernel(original_samples, noise, timesteps):
    batch = original_samples.shape[0]
    rest = original_samples.shape[1:]
    rows = int(np.prod(rest))
    ndim = original_samples.ndim
    to_batch_minor = tuple(range(1, ndim)) + (0,)
    from_batch_minor = (ndim - 1,) + tuple(range(ndim - 1))

    coefs = _make_sc_gather(batch)(_TABLE_A, _TABLE_B, timesteps.astype(jnp.int32))
    xt = original_samples.transpose(to_batch_minor).reshape(rows, batch)
    nt = noise.transpose(to_batch_minor).reshape(rows, batch)
    out = _combine(coefs, xt, nt, block_r=1024)
    return out.reshape(rest + (batch,)).transpose(from_batch_minor)


# speedup vs baseline: 1.0525x; 1.0525x over previous
import functools

import jax
import jax.numpy as jnp
import numpy as np
from jax import lax
from jax.experimental import pallas as pl
from jax.experimental.pallas import tpu as pltpu
from jax.experimental.pallas import tpu_sc as plsc

NUM_TIMESTEPS = 1000


def _make_tables() -> tuple[np.ndarray, np.ndarray]:
    s = np.float32(0.0001)
    x = np.linspace(0.0, float(NUM_TIMESTEPS), NUM_TIMESTEPS + 1, dtype=np.float32)
    acp = np.cos((x / NUM_TIMESTEPS + s) / (1 + s) * np.float32(np.pi) * 0.5,
                 dtype=np.float32) ** 2
    acp = acp / acp[0]
    betas = (1.0 - acp[1:] / acp[:-1]).astype(np.float32)
    betas = np.clip(betas, np.float32(0.02), np.float32(0.02))
    alphas = (1.0 - betas).astype(np.float32)
    acp2 = np.cumprod(alphas, dtype=np.float32)
    ta = np.zeros((1024,), dtype=np.float32)
    tb = np.zeros((1024,), dtype=np.float32)
    ta[:NUM_TIMESTEPS] = np.sqrt(acp2)
    tb[:NUM_TIMESTEPS] = np.sqrt(np.float32(1.0) - acp2)
    return ta, tb


_TABLE_A, _TABLE_B = _make_tables()


@functools.cache
def _make_sc_gather(batch: int):
    info = plsc.get_sparse_core_info()
    num_cores = info.num_cores
    num_workers = num_cores * info.num_subcores
    b_per_w = batch // num_workers
    mesh = plsc.VectorSubcoreMesh(core_axis_name="c", subcore_axis_name="s")

    @functools.partial(
        pl.kernel,
        mesh=mesh,
        out_type=jax.ShapeDtypeStruct((2 * batch,), jnp.float32),
        scratch_types=[
            pltpu.VMEM((b_per_w,), jnp.int32),
            pltpu.VMEM((b_per_w,), jnp.float32),
            pltpu.VMEM((b_per_w,), jnp.float32),
            pltpu.SemaphoreType.DMA,
            pltpu.SemaphoreType.DMA,
        ],
    )
    def gather(ta_hbm, tb_hbm, ts_hbm, out_hbm, idx_v, a_v, b_v, sem_a, sem_b):
        wid = lax.axis_index("s") * num_cores + lax.axis_index("c")
        base = wid * b_per_w
        pltpu.sync_copy(ts_hbm.at[pl.ds(base, b_per_w)], idx_v)
        ca = pltpu.async_copy(ta_hbm.at[idx_v], a_v, sem_a)
        cb = pltpu.async_copy(tb_hbm.at[idx_v], b_v, sem_b)
        ca.wait()
        cb.wait()
        pltpu.sync_copy(a_v, out_hbm.at[pl.ds(base, b_per_w)])
        pltpu.sync_copy(b_v, out_hbm.at[pl.ds(batch + base, b_per_w)])

    return gather


def _combine_body(coef_ref, x_ref, n_ref, o_ref):
    batch = x_ref.shape[1]
    c = coef_ref[...]
    a = c[0:batch].reshape(1, batch)
    b = c[batch:2 * batch].reshape(1, batch)
    o_ref[...] = a * x_ref[...] + b * n_ref[...]


def _combine(coefs, xt, nt, block_r: int, interpret: bool = False):
    rows, batch = xt.shape
    return pl.pallas_call(
        _combine_body,
        grid=(rows // block_r,),
        in_specs=[
            pl.BlockSpec((2 * batch,), lambda i: (0,)),
            pl.BlockSpec((block_r, batch), lambda i: (i, 0)),
            pl.BlockSpec((block_r, batch), lambda i: (i, 0)),
        ],
        out_specs=pl.BlockSpec((block_r, batch), lambda i: (i, 0)),
        out_shape=jax.ShapeDtypeStruct((rows, batch), jnp.float32),
        interpret=interpret,
    )(coefs, xt, nt)


def kernel(original_samples, noise, timesteps):
    batch = original_samples.shape[0]
    rest = original_samples.shape[1:]
    rows = int(np.prod(rest))
    ndim = original_samples.ndim
    to_batch_minor = tuple(range(1, ndim)) + (0,)
    from_batch_minor = (ndim - 1,) + tuple(range(ndim - 1))

    coefs = _make_sc_gather(batch)(_TABLE_A, _TABLE_B, timesteps.astype(jnp.int32))
    xt = original_samples.transpose(to_batch_minor).reshape(rows, batch)
    nt = noise.transpose(to_batch_minor).reshape(rows, batch)
    out = _combine(coefs, xt, nt, block_r=2048)
    return out.reshape(rest + (batch,)).transpose(from_batch_minor)
